# baseline (speedup 1.0000x reference)
import functools

import jax
import jax.numpy as jnp
from jax import lax
from jax.experimental import pallas as pl
from jax.experimental.pallas import tpu as pltpu
from jax.experimental.pallas import tpu_sc as plsc

_NC = 2
_NS = 16
_NW = _NC * _NS
_CHUNK = 128
_NBUF = 8


def kernel(inputs, table):
    B, F = inputs.shape
    V, D = table.shape
    N = B * F
    assert N % (_NW * _CHUNK) == 0
    per_w = N // _NW
    C = per_w // _CHUNK
    assert (C - _NBUF) % _NBUF == 0

    idx2d = inputs.reshape(N // _CHUNK, _CHUNK).astype(jnp.int32)
    mesh = plsc.VectorSubcoreMesh(core_axis_name="c", subcore_axis_name="s")

    @functools.partial(
        pl.kernel,
        mesh=mesh,
        out_type=jax.ShapeDtypeStruct((N, D), jnp.float32),
        compiler_params=pltpu.CompilerParams(use_tc_tiling_on_sc=False),
        scratch_types=(
            [
                pltpu.VMEM((C, _CHUNK), jnp.int32),
                pltpu.VMEM((_NBUF, _CHUNK, D), jnp.float32),
            ]
            + [pltpu.SemaphoreType.DMA] * (2 * _NBUF)
        ),
    )
    def run(table_hbm, idx_hbm, out_hbm, idx_v, rows_v, *sems):
        gsem = sems[:_NBUF]
        osem = sems[_NBUF:]
        wid = lax.axis_index("s") * _NC + lax.axis_index("c")
        cbase = wid * C

        pltpu.sync_copy(idx_hbm.at[pl.ds(cbase, C)], idx_v)

        def start_gather(j, b):
            pltpu.async_copy(table_hbm.at[idx_v.at[j]], rows_v.at[b], gsem[b])

        def wait_gather(j, b):
            pltpu.make_async_copy(
                table_hbm.at[idx_v.at[j]], rows_v.at[b], gsem[b]
            ).wait()

        def start_out(j, b):
            pltpu.async_copy(
                rows_v.at[b],
                out_hbm.at[pl.ds((cbase + j) * _CHUNK, _CHUNK)],
                osem[b],
            )

        def wait_out(j, b):
            pltpu.make_async_copy(
                rows_v.at[b],
                out_hbm.at[pl.ds((cbase + j) * _CHUNK, _CHUNK)],
                osem[b],
            ).wait()

        for b in range(_NBUF):
            start_gather(b, b)

        def outer(i, carry):
            g = i * _NBUF
            for b in range(_NBUF):
                j = g + b
                wait_gather(j, b)
                start_out(j, b)
                wait_out(j, b)
                start_gather(j + _NBUF, b)
            return carry

        lax.fori_loop(0, (C - _NBUF) // _NBUF, outer, 0)

        for b in range(_NBUF):
            j = C - _NBUF + b
            wait_gather(j, b)
            start_out(j, b)
            wait_out(j, b)

    out = run(table, idx2d)
    return out.reshape(B, F, D)
